# baseline (speedup 1.0000x reference)
import functools

import jax
import jax.numpy as jnp
from jax import lax
from jax.experimental import pallas as pl
from jax.experimental.pallas import tpu as pltpu
from jax.experimental.pallas import tpu_sc as plsc

N = 10000
K = 16
C = 64
NPAD = 10240
BQ = 128
CW = 256
NC = NPAD // CW
BIGI = NPAD

NW = 32
RPW = NPAD // NW
GB = 8
NG = RPW // GB


def _knn_body(qx_ref, qy_ref, qz_ref, px_ref, py_ref, pz_ref, idx_ref, d_ref):
    qx = qx_ref[...]
    qy = qy_ref[...]
    qz = qz_ref[...]
    lane = lax.broadcasted_iota(jnp.int32, (BQ, CW), 1)
    lane16 = lax.broadcasted_iota(jnp.int32, (BQ, K), 1)

    idx_acc = jnp.full((BQ, K), 0, jnp.int32)
    am_prev = jnp.full((BQ, 1), -1, jnp.int32)

    for k in range(K):
        first = k == 0

        def scan_chunk(j, carry, _first=first, _amp=am_prev):
            m, am = carry
            if _first:
                dx = qx - px_ref[j]
                dy = qy - py_ref[j]
                dz = qz - pz_ref[j]
                blk = (dx * dx + dy * dy) + dz * dz
            else:
                blk = d_ref[j]
                blk = jnp.where(lane + j * CW == _amp, jnp.inf, blk)
            d_ref[j] = blk
            mc = jnp.min(blk, axis=1, keepdims=True)
            cand = jnp.where(blk == mc, lane + j * CW, BIGI)
            amc = jnp.min(cand, axis=1, keepdims=True)
            better = mc < m
            tie = mc == m
            am2 = jnp.where(better, amc,
                            jnp.where(tie, jnp.minimum(am, amc), am))
            return jnp.minimum(m, mc), am2

        m0 = jnp.full((BQ, 1), jnp.inf, jnp.float32)
        a0 = jnp.full((BQ, 1), BIGI, jnp.int32)
        _, am = lax.fori_loop(0, NC, scan_chunk, (m0, a0))
        idx_acc = jnp.where(lane16 == k, am, idx_acc)
        am_prev = am

    idx_ref[...] = idx_acc


def _knn(qx, qy, qz, px, py, pz):
    return pl.pallas_call(
        _knn_body,
        grid=(NPAD // BQ,),
        in_specs=[
            pl.BlockSpec((BQ, 1), lambda i: (i, 0)),
            pl.BlockSpec((BQ, 1), lambda i: (i, 0)),
            pl.BlockSpec((BQ, 1), lambda i: (i, 0)),
            pl.BlockSpec((NC, 1, CW), lambda i: (0, 0, 0)),
            pl.BlockSpec((NC, 1, CW), lambda i: (0, 0, 0)),
            pl.BlockSpec((NC, 1, CW), lambda i: (0, 0, 0)),
        ],
        out_specs=pl.BlockSpec((BQ, K), lambda i: (i, 0)),
        out_shape=jax.ShapeDtypeStruct((NPAD, K), jnp.int32),
        scratch_shapes=[pltpu.VMEM((NC, BQ, CW), jnp.float32)],
        compiler_params=pltpu.CompilerParams(
            dimension_semantics=("arbitrary",)),
    )(qx, qy, qz, px, py, pz)


def _feat_body(x_ref, wc_ref, w2_ref, b_ref, ax_ref, bm_ref):
    x = x_ref[...]
    a = jnp.dot(x, wc_ref[...],
                preferred_element_type=jnp.float32) + b_ref[...]
    ax_ref[...] = jnp.concatenate([a, x], axis=1)
    bm = jnp.dot(x, w2_ref[...], preferred_element_type=jnp.float32)
    bm_ref[...] = jnp.concatenate([bm, jnp.zeros_like(bm)], axis=1)


def _feat(x, wc, w2, b):
    br = 1024
    return pl.pallas_call(
        _feat_body,
        grid=(NPAD // br,),
        in_specs=[
            pl.BlockSpec((br, C), lambda i: (i, 0)),
            pl.BlockSpec((C, C), lambda i: (0, 0)),
            pl.BlockSpec((C, C), lambda i: (0, 0)),
            pl.BlockSpec((1, C), lambda i: (0, 0)),
        ],
        out_specs=[
            pl.BlockSpec((br, 2 * C), lambda i: (i, 0)),
            pl.BlockSpec((br, 2 * C), lambda i: (i, 0)),
        ],
        out_shape=[
            jax.ShapeDtypeStruct((NPAD, 2 * C), jnp.float32),
            jax.ShapeDtypeStruct((NPAD, 2 * C), jnp.float32),
        ],
        compiler_params=pltpu.CompilerParams(
            dimension_semantics=("arbitrary",)),
    )(x, wc, w2, b)


def _gm_body(idx_hbm, bm_hbm, ax_hbm, out_hbm, idx_v, ax_v, o_v, g_v, sem):
    wid = lax.axis_index("s") * 2 + lax.axis_index("c")
    base = wid * RPW
    pltpu.sync_copy(idx_hbm.at[pl.ds(base * K, RPW * K)], idx_v)
    pltpu.sync_copy(ax_hbm.at[pl.ds(base, RPW)], ax_v)

    def step(t, _):
        pltpu.async_copy(
            bm_hbm.at[idx_v.at[pl.ds(t * GB * K, GB * K)]], g_v, sem).wait()
        for r in range(GB):
            row = t * GB + r
            for c in range(C // 16):
                sl = pl.ds(c * 16, 16)
                acc = g_v[r * K, sl]
                for n in range(1, K):
                    acc = jnp.maximum(acc, g_v[r * K + n, sl])
                o_v[row, sl] = ax_v[row, pl.ds(C + c * 16, 16)] + jnp.maximum(
                    ax_v[row, sl] + acc, 0.0)
        return 0

    lax.fori_loop(0, NG, step, 0)
    pltpu.sync_copy(o_v, out_hbm.at[pl.ds(base, RPW)])


@functools.lru_cache(maxsize=1)
def _build_gather_max():
    return functools.partial(
        pl.kernel,
        mesh=plsc.VectorSubcoreMesh(core_axis_name="c", subcore_axis_name="s"),
        out_type=jax.ShapeDtypeStruct((NPAD, C), jnp.float32),
        scratch_types=[
            pltpu.VMEM((RPW * K,), jnp.int32),
            pltpu.VMEM((RPW, 2 * C), jnp.float32),
            pltpu.VMEM((RPW, C), jnp.float32),
            pltpu.VMEM((GB * K, 2 * C), jnp.float32),
            pltpu.SemaphoreType.DMA,
        ],
    )(_gm_body)


def kernel(pillar_features, voxel_coords, W, b):
    n = pillar_features.shape[0]
    pos = voxel_coords[:, 1:4]
    pad = NPAD - n
    posp = jnp.concatenate(
        [pos, jnp.full((pad, 3), 1e9, jnp.float32)], axis=0)
    qx = posp[:, 0:1]
    qy = posp[:, 1:2]
    qz = posp[:, 2:3]
    px = posp[:, 0].reshape(NC, 1, CW)
    py = posp[:, 1].reshape(NC, 1, CW)
    pz = posp[:, 2].reshape(NC, 1, CW)

    idx = _knn(qx, qy, qz, px, py, pz)

    xp = jnp.concatenate(
        [pillar_features, jnp.zeros((pad, C), jnp.float32)], axis=0)
    wc = W[:C] - W[C:]
    w2 = W[C:]
    ax, bm = _feat(xp, wc, w2, b.reshape(1, C))

    out = _build_gather_max()(idx.reshape(-1), bm, ax)
    return out[:n]


# speedup vs baseline: 1.5940x; 1.5940x over previous
import functools

import jax
import jax.numpy as jnp
from jax import lax
from jax.experimental import pallas as pl
from jax.experimental.pallas import tpu as pltpu
from jax.experimental.pallas import tpu_sc as plsc

N = 10000
K = 16
C = 64
NPAD = 10240
BQ = 128
CW = 256
NC = NPAD // CW
BIGI = NPAD

NW = 32
RPW = NPAD // NW
GB = 8
NG = RPW // GB


def _knn_body(qx_ref, qy_ref, qz_ref, px_ref, py_ref, pz_ref, idx_ref,
              d_ref, ip_ref):
    qx = qx_ref[...]
    qy = qy_ref[...]
    qz = qz_ref[...]
    lane = lax.broadcasted_iota(jnp.int32, (BQ, CW), 1)
    lane16 = lax.broadcasted_iota(jnp.int32, (BQ, K), 1)
    lanec = lax.broadcasted_iota(jnp.int32, (BQ, NC), 1)
    BIG2 = jnp.int32(1 << 30)

    def fill(j, mc_acc):
        dx = qx - px_ref[j]
        dy = qy - py_ref[j]
        dz = qz - pz_ref[j]
        blk = (dx * dx + dy * dy) + dz * dz
        d_ref[j] = blk
        ip_ref[j] = jnp.full((BQ, 1), -1, jnp.int32)
        mc = jnp.min(blk, axis=1, keepdims=True)
        return jnp.where(lanec == j, mc, mc_acc)

    mcs = lax.fori_loop(
        0, NC, fill, jnp.full((BQ, NC), jnp.inf, jnp.float32))

    t0 = jnp.zeros((BQ, 1), jnp.float32)
    for _ in range(K):
        t0 = jnp.min(mcs, axis=1, keepdims=True)
        hit = jnp.where(mcs == t0, lanec, BIG2)
        ah = jnp.min(hit, axis=1, keepdims=True)
        mcs = jnp.where(lanec == ah, jnp.inf, mcs)

    def sweep_cond(carry):
        return carry[1] > 0

    def sweep_body(carry):
        s, _, tv, ti = carry

        def chunk_body(j, cc):
            cnt, cvs, cis = cc
            blk = d_ref[j]
            ip = ip_ref[j]
            cand = (blk <= t0) & (lane > ip)
            pos = jnp.min(jnp.where(cand, lane, BIGI), axis=1, keepdims=True)
            fnd = pos < BIGI
            val = jnp.min(jnp.where(lane == pos, blk, jnp.inf),
                          axis=1, keepdims=True)
            ip_ref[j] = jnp.where(fnd, pos, ip)
            gidx = jnp.where(fnd, pos + j * CW, BIGI + j)
            cvs = jnp.where(lanec == j, val, cvs)
            cis = jnp.where(lanec == j, gidx, cis)
            return cnt + jnp.sum(fnd.astype(jnp.int32)), cvs, cis

        cnt, cvs, cis = lax.fori_loop(
            0, NC, chunk_body,
            (jnp.zeros((), jnp.int32),
             jnp.full((BQ, NC), jnp.inf, jnp.float32),
             jnp.full((BQ, NC), BIGI, jnp.int32)))

        comb_v = jnp.concatenate([tv, cvs], axis=1)
        comb_i = jnp.concatenate([ti, cis], axis=1)
        tv2 = tv
        ti2 = ti
        for k in range(K):
            m = jnp.min(comb_v, axis=1, keepdims=True)
            ai = jnp.min(jnp.where(comb_v == m, comb_i, BIG2),
                         axis=1, keepdims=True)
            tv2 = jnp.where(lane16 == k, m, tv2)
            ti2 = jnp.where(lane16 == k, ai, ti2)
            kill = comb_i == ai
            comb_v = jnp.where(kill, jnp.inf, comb_v)
            comb_i = jnp.where(kill, BIG2, comb_i)
        return s + 1, cnt, tv2, ti2

    _, _, _, ti = lax.while_loop(
        sweep_cond, sweep_body,
        (jnp.zeros((), jnp.int32), jnp.ones((), jnp.int32),
         jnp.full((BQ, K), jnp.inf, jnp.float32),
         jnp.full((BQ, K), BIGI, jnp.int32)))
    idx_ref[...] = ti


def _knn(qx, qy, qz, px, py, pz):
    return pl.pallas_call(
        _knn_body,
        grid=(NPAD // BQ,),
        in_specs=[
            pl.BlockSpec((BQ, 1), lambda i: (i, 0)),
            pl.BlockSpec((BQ, 1), lambda i: (i, 0)),
            pl.BlockSpec((BQ, 1), lambda i: (i, 0)),
            pl.BlockSpec((NC, 1, CW), lambda i: (0, 0, 0)),
            pl.BlockSpec((NC, 1, CW), lambda i: (0, 0, 0)),
            pl.BlockSpec((NC, 1, CW), lambda i: (0, 0, 0)),
        ],
        out_specs=pl.BlockSpec((BQ, K), lambda i: (i, 0)),
        out_shape=jax.ShapeDtypeStruct((NPAD, K), jnp.int32),
        scratch_shapes=[pltpu.VMEM((NC, BQ, CW), jnp.float32),
                        pltpu.VMEM((NC, BQ, 1), jnp.int32)],
        compiler_params=pltpu.CompilerParams(
            dimension_semantics=("arbitrary",)),
    )(qx, qy, qz, px, py, pz)


def _feat_body(x_ref, wc_ref, w2_ref, b_ref, ax_ref, bm_ref):
    x = x_ref[...]
    a = jnp.dot(x, wc_ref[...],
                preferred_element_type=jnp.float32) + b_ref[...]
    ax_ref[...] = jnp.concatenate([a, x], axis=1)
    bm = jnp.dot(x, w2_ref[...], preferred_element_type=jnp.float32)
    bm_ref[...] = jnp.concatenate([bm, jnp.zeros_like(bm)], axis=1)


def _feat(x, wc, w2, b):
    br = 1024
    return pl.pallas_call(
        _feat_body,
        grid=(NPAD // br,),
        in_specs=[
            pl.BlockSpec((br, C), lambda i: (i, 0)),
            pl.BlockSpec((C, C), lambda i: (0, 0)),
            pl.BlockSpec((C, C), lambda i: (0, 0)),
            pl.BlockSpec((1, C), lambda i: (0, 0)),
        ],
        out_specs=[
            pl.BlockSpec((br, 2 * C), lambda i: (i, 0)),
            pl.BlockSpec((br, 2 * C), lambda i: (i, 0)),
        ],
        out_shape=[
            jax.ShapeDtypeStruct((NPAD, 2 * C), jnp.float32),
            jax.ShapeDtypeStruct((NPAD, 2 * C), jnp.float32),
        ],
        compiler_params=pltpu.CompilerParams(
            dimension_semantics=("arbitrary",)),
    )(x, wc, w2, b)


def _gm_body(idx_hbm, bm_hbm, ax_hbm, out_hbm, idx_v, ax_v, o_v, g_v, sem):
    wid = lax.axis_index("s") * 2 + lax.axis_index("c")
    base = wid * RPW
    pltpu.sync_copy(idx_hbm.at[pl.ds(base * K, RPW * K)], idx_v)
    pltpu.sync_copy(ax_hbm.at[pl.ds(base, RPW)], ax_v)

    def step(t, _):
        pltpu.async_copy(
            bm_hbm.at[idx_v.at[pl.ds(t * GB * K, GB * K)]], g_v, sem).wait()
        for r in range(GB):
            row = t * GB + r
            for c in range(C // 16):
                sl = pl.ds(c * 16, 16)
                acc = g_v[r * K, sl]
                for n in range(1, K):
                    acc = jnp.maximum(acc, g_v[r * K + n, sl])
                o_v[row, sl] = ax_v[row, pl.ds(C + c * 16, 16)] + jnp.maximum(
                    ax_v[row, sl] + acc, 0.0)
        return 0

    lax.fori_loop(0, NG, step, 0)
    pltpu.sync_copy(o_v, out_hbm.at[pl.ds(base, RPW)])


@functools.lru_cache(maxsize=1)
def _build_gather_max():
    return functools.partial(
        pl.kernel,
        mesh=plsc.VectorSubcoreMesh(core_axis_name="c", subcore_axis_name="s"),
        out_type=jax.ShapeDtypeStruct((NPAD, C), jnp.float32),
        scratch_types=[
            pltpu.VMEM((RPW * K,), jnp.int32),
            pltpu.VMEM((RPW, 2 * C), jnp.float32),
            pltpu.VMEM((RPW, C), jnp.float32),
            pltpu.VMEM((GB * K, 2 * C), jnp.float32),
            pltpu.SemaphoreType.DMA,
        ],
    )(_gm_body)


def kernel(pillar_features, voxel_coords, W, b):
    n = pillar_features.shape[0]
    pos = voxel_coords[:, 1:4]
    pad = NPAD - n
    posq = jnp.concatenate(
        [pos, jnp.broadcast_to(pos[0], (pad, 3))], axis=0)
    posp = jnp.concatenate(
        [pos, jnp.full((pad, 3), 1e9, jnp.float32)], axis=0)
    qx = posq[:, 0:1]
    qy = posq[:, 1:2]
    qz = posq[:, 2:3]
    px = posp[:, 0].reshape(NC, 1, CW)
    py = posp[:, 1].reshape(NC, 1, CW)
    pz = posp[:, 2].reshape(NC, 1, CW)

    idx = _knn(qx, qy, qz, px, py, pz)

    xp = jnp.concatenate(
        [pillar_features, jnp.zeros((pad, C), jnp.float32)], axis=0)
    wc = W[:C] - W[C:]
    w2 = W[C:]
    ax, bm = _feat(xp, wc, w2, b.reshape(1, C))

    out = _build_gather_max()(idx.reshape(-1), bm, ax)
    return out[:n]


# speedup vs baseline: 4.0985x; 2.5712x over previous
import functools

import jax
import jax.numpy as jnp
from jax import lax
from jax.experimental import pallas as pl
from jax.experimental.pallas import tpu as pltpu
from jax.experimental.pallas import tpu_sc as plsc

N = 10000
K = 16
C = 64
NPAD = 10240
BQ = 128
CW = 256
NC = NPAD // CW
BIGI = NPAD

NW = 32
RPW = NPAD // NW
GB = 8
NG = RPW // GB


NEX = 4
FBIG = float(BIGI)
FBIG2 = float(1 << 24)


def _knn_body(qx_ref, qy_ref, qz_ref, px_ref, py_ref, pz_ref, idx_ref,
              d_ref, ip_ref):
    qx = qx_ref[...]
    qy = qy_ref[...]
    qz = qz_ref[...]
    lane = lax.broadcasted_iota(
        jnp.int32, (BQ, CW), 1).astype(jnp.float32)
    lane16 = lax.broadcasted_iota(
        jnp.int32, (BQ, K), 1).astype(jnp.float32)
    lanec = lax.broadcasted_iota(
        jnp.int32, (BQ, NC), 1).astype(jnp.float32)
    inf = jnp.inf

    def fill(j, mc_acc):
        dx = qx - px_ref[j]
        dy = qy - py_ref[j]
        dz = qz - pz_ref[j]
        blk = (dx * dx + dy * dy) + dz * dz
        d_ref[j] = blk
        mc = jnp.min(blk, axis=1, keepdims=True)
        return jnp.where(lanec == j.astype(jnp.float32), mc, mc_acc)

    mcs = lax.fori_loop(
        0, NC, fill, jnp.full((BQ, NC), inf, jnp.float32))

    t0 = jnp.zeros((BQ, 1), jnp.float32)
    for _ in range(K):
        t0 = jnp.min(mcs, axis=1, keepdims=True)
        ah = jnp.min(jnp.where(mcs == t0, lanec, FBIG2),
                     axis=1, keepdims=True)
        mcs = jnp.where(lanec == ah, inf, mcs)

    def extract(j, cc):
        rem, vs, is_ = cc
        jf = j.astype(jnp.float32)
        blk = d_ref[j]
        cand = blk <= t0
        cnt = jnp.sum(cand.astype(jnp.float32), axis=1, keepdims=True)
        pos = jnp.full((BQ, 1), -1.0, jnp.float32)
        vs2, is2 = [], []
        for _ in range(NEX):
            pos = jnp.min(jnp.where(cand & (lane > pos), lane, FBIG),
                          axis=1, keepdims=True)
            val = jnp.min(jnp.where(lane == pos, blk, inf),
                          axis=1, keepdims=True)
            gid = jnp.where(pos < FBIG, pos + jf * CW, FBIG + jf)
            vs2.append(val)
            is2.append(gid)
        ip_ref[j] = pos
        rem = rem + jnp.sum(jnp.maximum(cnt - NEX, 0.0))
        sel = lanec == jf
        vs = [jnp.where(sel, v2, v) for v, v2 in zip(vs, vs2)]
        is_ = [jnp.where(sel, i2, i) for i, i2 in zip(is_, is2)]
        return rem, vs, is_

    rem, vs, is_ = lax.fori_loop(
        0, NC, extract,
        (jnp.zeros((), jnp.float32),
         [jnp.full((BQ, NC), inf, jnp.float32) for _ in range(NEX)],
         [jnp.full((BQ, NC), FBIG, jnp.float32) for _ in range(NEX)]))

    def top16(comb_v, comb_i, tv, ti):
        for k in range(K):
            m = jnp.min(comb_v, axis=1, keepdims=True)
            ai = jnp.min(jnp.where(comb_v == m, comb_i, FBIG2),
                         axis=1, keepdims=True)
            tv = jnp.where(lane16 == k, m, tv)
            ti = jnp.where(lane16 == k, ai, ti)
            kill = comb_i == ai
            comb_v = jnp.where(kill, inf, comb_v)
            comb_i = jnp.where(kill, FBIG2, comb_i)
        return tv, ti

    tv, ti = top16(
        jnp.concatenate(vs, axis=1), jnp.concatenate(is_, axis=1),
        jnp.full((BQ, K), inf, jnp.float32),
        jnp.full((BQ, K), FBIG2, jnp.float32))

    def sweep_cond(carry):
        return carry[0] > 0.5

    def sweep_body(carry):
        rem, tv, ti = carry

        def chunk_body(j, cc):
            fc, cvs, cis = cc
            jf = j.astype(jnp.float32)
            blk = d_ref[j]
            ip = ip_ref[j]
            cand = (blk <= t0) & (lane > ip)
            pos = jnp.min(jnp.where(cand, lane, FBIG),
                          axis=1, keepdims=True)
            fnd = pos < FBIG
            val = jnp.min(jnp.where(lane == pos, blk, inf),
                          axis=1, keepdims=True)
            ip_ref[j] = jnp.where(fnd, pos, ip)
            gid = jnp.where(fnd, pos + jf * CW, FBIG + jf)
            sel = lanec == jf
            cvs = jnp.where(sel, val, cvs)
            cis = jnp.where(sel, gid, cis)
            return fc + jnp.sum(fnd.astype(jnp.float32)), cvs, cis

        fc, cvs, cis = lax.fori_loop(
            0, NC, chunk_body,
            (jnp.zeros((), jnp.float32),
             jnp.full((BQ, NC), inf, jnp.float32),
             jnp.full((BQ, NC), FBIG, jnp.float32)))
        tv, ti = top16(jnp.concatenate([tv, cvs], axis=1),
                       jnp.concatenate([ti, cis], axis=1), tv, ti)
        return rem - fc, tv, ti

    _, _, ti = lax.while_loop(sweep_cond, sweep_body, (rem, tv, ti))
    idx_ref[...] = ti.astype(jnp.int32)


def _knn(qx, qy, qz, px, py, pz):
    return pl.pallas_call(
        _knn_body,
        grid=(NPAD // BQ,),
        in_specs=[
            pl.BlockSpec((BQ, 1), lambda i: (i, 0)),
            pl.BlockSpec((BQ, 1), lambda i: (i, 0)),
            pl.BlockSpec((BQ, 1), lambda i: (i, 0)),
            pl.BlockSpec((NC, 1, CW), lambda i: (0, 0, 0)),
            pl.BlockSpec((NC, 1, CW), lambda i: (0, 0, 0)),
            pl.BlockSpec((NC, 1, CW), lambda i: (0, 0, 0)),
        ],
        out_specs=pl.BlockSpec((BQ, K), lambda i: (i, 0)),
        out_shape=jax.ShapeDtypeStruct((NPAD, K), jnp.int32),
        scratch_shapes=[pltpu.VMEM((NC, BQ, CW), jnp.float32),
                        pltpu.VMEM((NC, BQ, 1), jnp.float32)],
        compiler_params=pltpu.CompilerParams(
            dimension_semantics=("arbitrary",)),
    )(qx, qy, qz, px, py, pz)


def _feat_body(x_ref, wc_ref, w2_ref, b_ref, ax_ref, bm_ref):
    x = x_ref[...]
    a = jnp.dot(x, wc_ref[...],
                preferred_element_type=jnp.float32) + b_ref[...]
    ax_ref[...] = jnp.concatenate([a, x], axis=1)
    bm = jnp.dot(x, w2_ref[...], preferred_element_type=jnp.float32)
    bm_ref[...] = jnp.concatenate([bm, jnp.zeros_like(bm)], axis=1)


def _feat(x, wc, w2, b):
    br = 1024
    return pl.pallas_call(
        _feat_body,
        grid=(NPAD // br,),
        in_specs=[
            pl.BlockSpec((br, C), lambda i: (i, 0)),
            pl.BlockSpec((C, C), lambda i: (0, 0)),
            pl.BlockSpec((C, C), lambda i: (0, 0)),
            pl.BlockSpec((1, C), lambda i: (0, 0)),
        ],
        out_specs=[
            pl.BlockSpec((br, 2 * C), lambda i: (i, 0)),
            pl.BlockSpec((br, 2 * C), lambda i: (i, 0)),
        ],
        out_shape=[
            jax.ShapeDtypeStruct((NPAD, 2 * C), jnp.float32),
            jax.ShapeDtypeStruct((NPAD, 2 * C), jnp.float32),
        ],
        compiler_params=pltpu.CompilerParams(
            dimension_semantics=("arbitrary",)),
    )(x, wc, w2, b)


def _gm_body(idx_hbm, bm_hbm, ax_hbm, out_hbm, idx_v, ax_v, o_v, g_v, sem):
    wid = lax.axis_index("s") * 2 + lax.axis_index("c")
    base = wid * RPW
    pltpu.sync_copy(idx_hbm.at[pl.ds(base * K, RPW * K)], idx_v)
    pltpu.sync_copy(ax_hbm.at[pl.ds(base, RPW)], ax_v)

    def step(t, _):
        pltpu.async_copy(
            bm_hbm.at[idx_v.at[pl.ds(t * GB * K, GB * K)]], g_v, sem).wait()
        for r in range(GB):
            row = t * GB + r
            for c in range(C // 16):
                sl = pl.ds(c * 16, 16)
                acc = g_v[r * K, sl]
                for n in range(1, K):
                    acc = jnp.maximum(acc, g_v[r * K + n, sl])
                o_v[row, sl] = ax_v[row, pl.ds(C + c * 16, 16)] + jnp.maximum(
                    ax_v[row, sl] + acc, 0.0)
        return 0

    lax.fori_loop(0, NG, step, 0)
    pltpu.sync_copy(o_v, out_hbm.at[pl.ds(base, RPW)])


@functools.lru_cache(maxsize=1)
def _build_gather_max():
    return functools.partial(
        pl.kernel,
        mesh=plsc.VectorSubcoreMesh(core_axis_name="c", subcore_axis_name="s"),
        out_type=jax.ShapeDtypeStruct((NPAD, C), jnp.float32),
        scratch_types=[
            pltpu.VMEM((RPW * K,), jnp.int32),
            pltpu.VMEM((RPW, 2 * C), jnp.float32),
            pltpu.VMEM((RPW, C), jnp.float32),
            pltpu.VMEM((GB * K, 2 * C), jnp.float32),
            pltpu.SemaphoreType.DMA,
        ],
    )(_gm_body)


def kernel(pillar_features, voxel_coords, W, b):
    n = pillar_features.shape[0]
    pos = voxel_coords[:, 1:4]
    pad = NPAD - n
    posq = jnp.concatenate(
        [pos, jnp.broadcast_to(pos[0], (pad, 3))], axis=0)
    posp = jnp.concatenate(
        [pos, jnp.full((pad, 3), 1e9, jnp.float32)], axis=0)
    qx = posq[:, 0:1]
    qy = posq[:, 1:2]
    qz = posq[:, 2:3]
    px = posp[:, 0].reshape(NC, 1, CW)
    py = posp[:, 1].reshape(NC, 1, CW)
    pz = posp[:, 2].reshape(NC, 1, CW)

    idx = _knn(qx, qy, qz, px, py, pz)

    xp = jnp.concatenate(
        [pillar_features, jnp.zeros((pad, C), jnp.float32)], axis=0)
    wc = W[:C] - W[C:]
    w2 = W[C:]
    ax, bm = _feat(xp, wc, w2, b.reshape(1, C))

    out = _build_gather_max()(idx.reshape(-1), bm, ax)
    return out[:n]


# speedup vs baseline: 7.8057x; 1.9045x over previous
import functools

import jax
import jax.numpy as jnp
from jax import lax
from jax.experimental import pallas as pl
from jax.experimental.pallas import tpu as pltpu
from jax.experimental.pallas import tpu_sc as plsc

N = 10000
K = 16
C = 64
NPAD = 10240
BQ = 256
CW = 256
NC = NPAD // CW
BIGI = NPAD

NW = 32
RPW = NPAD // NW
GB = 8
NG = RPW // GB


NEX = 5
FBIG = float(BIGI)
FBIG2 = float(1 << 24)


def _knn_body(qx_ref, qy_ref, qz_ref, px_ref, py_ref, pz_ref, idx_ref,
              d_ref, ip_ref):
    qx = qx_ref[...]
    qy = qy_ref[...]
    qz = qz_ref[...]
    lane = lax.broadcasted_iota(
        jnp.int32, (BQ, CW), 1).astype(jnp.float32)
    lane16 = lax.broadcasted_iota(
        jnp.int32, (BQ, K), 1).astype(jnp.float32)
    lanec = lax.broadcasted_iota(
        jnp.int32, (BQ, NC), 1).astype(jnp.float32)
    inf = jnp.inf

    def fill(j, mc_acc):
        for u in range(2):
            jj = 2 * j + u
            dx = qx - px_ref[jj]
            dy = qy - py_ref[jj]
            dz = qz - pz_ref[jj]
            blk = (dx * dx + dy * dy) + dz * dz
            d_ref[jj] = blk
            mc = jnp.min(blk, axis=1, keepdims=True)
            mc_acc = jnp.where(lanec == jj.astype(jnp.float32), mc, mc_acc)
        return mc_acc

    mcs = lax.fori_loop(
        0, NC // 2, fill, jnp.full((BQ, NC), inf, jnp.float32))

    t0 = jnp.zeros((BQ, 1), jnp.float32)
    for _ in range(K):
        t0 = jnp.min(mcs, axis=1, keepdims=True)
        ah = jnp.min(jnp.where(mcs == t0, lanec, FBIG2),
                     axis=1, keepdims=True)
        mcs = jnp.where(lanec == ah, inf, mcs)

    def extract(j, cc):
        rem, vs, is_ = cc
        for u in range(2):
            jj = 2 * j + u
            jf = jj.astype(jnp.float32)
            blk = d_ref[jj]
            cand = blk <= t0
            cnt = jnp.sum(cand.astype(jnp.float32), axis=1, keepdims=True)
            pos = jnp.full((BQ, 1), -1.0, jnp.float32)
            vs2, is2 = [], []
            for _ in range(NEX):
                pos = jnp.min(jnp.where(cand & (lane > pos), lane, FBIG),
                              axis=1, keepdims=True)
                val = jnp.min(jnp.where(lane == pos, blk, inf),
                              axis=1, keepdims=True)
                gid = jnp.where(pos < FBIG, pos + jf * CW, FBIG + jf)
                vs2.append(val)
                is2.append(gid)
            ip_ref[jj] = pos
            rem = rem + jnp.sum(jnp.maximum(cnt - NEX, 0.0))
            sel = lanec == jf
            vs = [jnp.where(sel, v2, v) for v, v2 in zip(vs, vs2)]
            is_ = [jnp.where(sel, i2, i) for i, i2 in zip(is_, is2)]
        return rem, vs, is_

    rem, vs, is_ = lax.fori_loop(
        0, NC // 2, extract,
        (jnp.zeros((), jnp.float32),
         [jnp.full((BQ, NC), inf, jnp.float32) for _ in range(NEX)],
         [jnp.full((BQ, NC), FBIG, jnp.float32) for _ in range(NEX)]))

    def top16(comb_v, comb_i, tv, ti):
        for k in range(K):
            m = jnp.min(comb_v, axis=1, keepdims=True)
            ai = jnp.min(jnp.where(comb_v == m, comb_i, FBIG2),
                         axis=1, keepdims=True)
            tv = jnp.where(lane16 == k, m, tv)
            ti = jnp.where(lane16 == k, ai, ti)
            kill = comb_i == ai
            comb_v = jnp.where(kill, inf, comb_v)
            comb_i = jnp.where(kill, FBIG2, comb_i)
        return tv, ti

    tv, ti = top16(
        jnp.concatenate(vs, axis=1), jnp.concatenate(is_, axis=1),
        jnp.full((BQ, K), inf, jnp.float32),
        jnp.full((BQ, K), FBIG2, jnp.float32))

    def sweep_cond(carry):
        return carry[0] > 0.5

    def sweep_body(carry):
        rem, tv, ti = carry

        def chunk_body(j, cc):
            fc, cvs, cis = cc
            jf = j.astype(jnp.float32)
            blk = d_ref[j]
            ip = ip_ref[j]
            cand = (blk <= t0) & (lane > ip)
            pos = jnp.min(jnp.where(cand, lane, FBIG),
                          axis=1, keepdims=True)
            fnd = pos < FBIG
            val = jnp.min(jnp.where(lane == pos, blk, inf),
                          axis=1, keepdims=True)
            ip_ref[j] = jnp.where(fnd, pos, ip)
            gid = jnp.where(fnd, pos + jf * CW, FBIG + jf)
            sel = lanec == jf
            cvs = jnp.where(sel, val, cvs)
            cis = jnp.where(sel, gid, cis)
            return fc + jnp.sum(fnd.astype(jnp.float32)), cvs, cis

        fc, cvs, cis = lax.fori_loop(
            0, NC, chunk_body,
            (jnp.zeros((), jnp.float32),
             jnp.full((BQ, NC), inf, jnp.float32),
             jnp.full((BQ, NC), FBIG, jnp.float32)))
        tv, ti = top16(jnp.concatenate([tv, cvs], axis=1),
                       jnp.concatenate([ti, cis], axis=1), tv, ti)
        return rem - fc, tv, ti

    _, _, ti = lax.while_loop(sweep_cond, sweep_body, (rem, tv, ti))
    idx_ref[...] = ti.astype(jnp.int32)


def _knn(qx, qy, qz, px, py, pz):
    return pl.pallas_call(
        _knn_body,
        grid=(NPAD // BQ,),
        in_specs=[
            pl.BlockSpec((BQ, 1), lambda i: (i, 0)),
            pl.BlockSpec((BQ, 1), lambda i: (i, 0)),
            pl.BlockSpec((BQ, 1), lambda i: (i, 0)),
            pl.BlockSpec((NC, 1, CW), lambda i: (0, 0, 0)),
            pl.BlockSpec((NC, 1, CW), lambda i: (0, 0, 0)),
            pl.BlockSpec((NC, 1, CW), lambda i: (0, 0, 0)),
        ],
        out_specs=pl.BlockSpec((BQ, K), lambda i: (i, 0)),
        out_shape=jax.ShapeDtypeStruct((NPAD, K), jnp.int32),
        scratch_shapes=[pltpu.VMEM((NC, BQ, CW), jnp.float32),
                        pltpu.VMEM((NC, BQ, 1), jnp.float32)],
        compiler_params=pltpu.CompilerParams(
            dimension_semantics=("arbitrary",)),
    )(qx, qy, qz, px, py, pz)


def _feat_body(x_ref, wc_ref, w2_ref, b_ref, ax_ref, bm_ref):
    x = x_ref[...]
    a = jnp.dot(x, wc_ref[...],
                preferred_element_type=jnp.float32) + b_ref[...]
    ax_ref[...] = jnp.concatenate([a, x], axis=1)
    bm = jnp.dot(x, w2_ref[...], preferred_element_type=jnp.float32)
    bm_ref[...] = jnp.concatenate([bm, jnp.zeros_like(bm)], axis=1)


def _feat(x, wc, w2, b):
    br = 1024
    return pl.pallas_call(
        _feat_body,
        grid=(NPAD // br,),
        in_specs=[
            pl.BlockSpec((br, C), lambda i: (i, 0)),
            pl.BlockSpec((C, C), lambda i: (0, 0)),
            pl.BlockSpec((C, C), lambda i: (0, 0)),
            pl.BlockSpec((1, C), lambda i: (0, 0)),
        ],
        out_specs=[
            pl.BlockSpec((br, 2 * C), lambda i: (i, 0)),
            pl.BlockSpec((br, 2 * C), lambda i: (i, 0)),
        ],
        out_shape=[
            jax.ShapeDtypeStruct((NPAD, 2 * C), jnp.float32),
            jax.ShapeDtypeStruct((NPAD, 2 * C), jnp.float32),
        ],
        compiler_params=pltpu.CompilerParams(
            dimension_semantics=("arbitrary",)),
    )(x, wc, w2, b)


def _gm_body(idx_hbm, bm_hbm, ax_hbm, out_hbm, idx_v, ax_v, o_v, g_v, sem):
    wid = lax.axis_index("s") * 2 + lax.axis_index("c")
    base = wid * RPW
    pltpu.sync_copy(idx_hbm.at[pl.ds(base * K, RPW * K)], idx_v)
    pltpu.sync_copy(ax_hbm.at[pl.ds(base, RPW)], ax_v)

    def step(t, _):
        pltpu.async_copy(
            bm_hbm.at[idx_v.at[pl.ds(t * GB * K, GB * K)]], g_v, sem).wait()
        for r in range(GB):
            row = t * GB + r
            for c in range(C // 16):
                sl = pl.ds(c * 16, 16)
                acc = g_v[r * K, sl]
                for n in range(1, K):
                    acc = jnp.maximum(acc, g_v[r * K + n, sl])
                o_v[row, sl] = ax_v[row, pl.ds(C + c * 16, 16)] + jnp.maximum(
                    ax_v[row, sl] + acc, 0.0)
        return 0

    lax.fori_loop(0, NG, step, 0)
    pltpu.sync_copy(o_v, out_hbm.at[pl.ds(base, RPW)])


@functools.lru_cache(maxsize=1)
def _build_gather_max():
    return functools.partial(
        pl.kernel,
        mesh=plsc.VectorSubcoreMesh(core_axis_name="c", subcore_axis_name="s"),
        out_type=jax.ShapeDtypeStruct((NPAD, C), jnp.float32),
        scratch_types=[
            pltpu.VMEM((RPW * K,), jnp.int32),
            pltpu.VMEM((RPW, 2 * C), jnp.float32),
            pltpu.VMEM((RPW, C), jnp.float32),
            pltpu.VMEM((GB * K, 2 * C), jnp.float32),
            pltpu.SemaphoreType.DMA,
        ],
    )(_gm_body)


def kernel(pillar_features, voxel_coords, W, b):
    n = pillar_features.shape[0]
    pos = voxel_coords[:, 1:4]
    pad = NPAD - n
    posq = jnp.concatenate(
        [pos, jnp.broadcast_to(pos[0], (pad, 3))], axis=0)
    posp = jnp.concatenate(
        [pos, jnp.full((pad, 3), 1e9, jnp.float32)], axis=0)
    qx = posq[:, 0:1]
    qy = posq[:, 1:2]
    qz = posq[:, 2:3]
    px = posp[:, 0].reshape(NC, 1, CW)
    py = posp[:, 1].reshape(NC, 1, CW)
    pz = posp[:, 2].reshape(NC, 1, CW)

    idx = _knn(qx, qy, qz, px, py, pz)

    xp = jnp.concatenate(
        [pillar_features, jnp.zeros((pad, C), jnp.float32)], axis=0)
    wc = W[:C] - W[C:]
    w2 = W[C:]
    ax, bm = _feat(xp, wc, w2, b.reshape(1, C))

    out = _build_gather_max()(idx.reshape(-1), bm, ax)
    return out[:n]
